# baseline (speedup 1.0000x reference)
import jax
import jax.numpy as jnp
from jax.experimental import pallas as pl
from jax.experimental.pallas import tpu as pltpu

N, D, H, O, E = 4096, 1024, 2048, 1024, 3
BN = 1024


def _moe_kernel(x_ref, gw1_ref, gb1_ref, gw2_ref, gb2_ref, gw3_ref, gb3_ref,
                ew1_ref, eb1_ref, ew2_ref, eb2_ref, out_ref, gates_ref,
                w_scr, x16_scr):
    e = pl.program_id(1)

    @pl.when(e == 0)
    def _gating():
        xb = x_ref[...]
        h1 = jax.nn.relu(
            jnp.dot(xb, gw1_ref[...], preferred_element_type=jnp.float32)
            + gb1_ref[...])
        h2 = jax.nn.relu(
            jnp.dot(h1, gw2_ref[...], preferred_element_type=jnp.float32)
            + gb2_ref[...])
        logits = jnp.dot(h2, gw3_ref[...],
                         preferred_element_type=jnp.float32) + gb3_ref[...]
        gates = jax.nn.softmax(logits, axis=-1)
        gates_ref[...] = gates

        g0, g1, g2 = gates[:, 0], gates[:, 1], gates[:, 2]
        drop2 = (g2 <= g0) & (g2 <= g1)
        drop1 = (~drop2) & (g1 <= g0) & (g1 <= g2)
        drop0 = (~drop2) & (~drop1)
        gmin = jnp.where(drop2, g2, jnp.where(drop1, g1, g0))
        denom = (g0 + g1 + g2) - gmin
        w_scr[0, :] = jnp.where(drop0, 0.0, g0) / denom
        w_scr[1, :] = jnp.where(drop1, 0.0, g1) / denom
        w_scr[2, :] = jnp.where(drop2, 0.0, g2) / denom
        x16_scr[...] = xb.astype(jnp.bfloat16)

    xb16 = x16_scr[...]
    h = jax.nn.relu(
        jnp.dot(xb16, ew1_ref[0], preferred_element_type=jnp.float32)
        + eb1_ref[0])
    o = jnp.dot(h.astype(jnp.bfloat16), ew2_ref[0],
                preferred_element_type=jnp.float32) + eb2_ref[0]
    contrib = w_scr[e].reshape(-1, 1) * o

    @pl.when(e == 0)
    def _init():
        out_ref[...] = contrib

    @pl.when(e != 0)
    def _accum():
        out_ref[...] = out_ref[...] + contrib


def kernel(x, gw1, gb1, gw2, gb2, gw3, gb3, ew1, eb1, ew2, eb2):
    ew1 = ew1.astype(jnp.bfloat16)
    ew2 = ew2.astype(jnp.bfloat16)
    eb1 = eb1.reshape(E, 1, H)
    eb2 = eb2.reshape(E, 1, O)
    gb1 = gb1.reshape(1, -1)
    gb2 = gb2.reshape(1, -1)
    gb3 = gb3.reshape(1, -1)

    grid = (N // BN, E)
    full = lambda b, e: (0, 0)
    out, gates = pl.pallas_call(
        _moe_kernel,
        grid=grid,
        in_specs=[
            pl.BlockSpec((BN, D), lambda b, e: (b, 0)),
            pl.BlockSpec((D, 256), full),
            pl.BlockSpec((1, 256), full),
            pl.BlockSpec((256, 128), full),
            pl.BlockSpec((1, 128), full),
            pl.BlockSpec((128, E), full),
            pl.BlockSpec((1, E), full),
            pl.BlockSpec((1, D, H), lambda b, e: (e, 0, 0)),
            pl.BlockSpec((1, 1, H), lambda b, e: (e, 0, 0)),
            pl.BlockSpec((1, H, O), lambda b, e: (e, 0, 0)),
            pl.BlockSpec((1, 1, O), lambda b, e: (e, 0, 0)),
        ],
        out_specs=[
            pl.BlockSpec((BN, O), lambda b, e: (b, 0)),
            pl.BlockSpec((BN, E), lambda b, e: (b, 0)),
        ],
        out_shape=[
            jax.ShapeDtypeStruct((N, O), jnp.float32),
            jax.ShapeDtypeStruct((N, E), jnp.float32),
        ],
        scratch_shapes=[
            pltpu.VMEM((E, BN), jnp.float32),
            pltpu.VMEM((BN, D), jnp.bfloat16),
        ],
    )(x, gw1, gb1, gw2, gb2, gw3, gb3, ew1, eb1, ew2, eb2)
    return out, gates
